# Initial kernel scaffold; baseline (speedup 1.0000x reference)
import jax
import jax.numpy as jnp
from jax.experimental import pallas as pl


def kernel(height_field):
    raise NotImplementedError("write your pallas kernel here")


# speedup vs baseline: 90.0099x; 90.0099x over previous
import jax
import jax.numpy as jnp
from jax.experimental import pallas as pl

IM_SIZE = 1024
RADIUS = 128
BATCH = 512
_PAD = -1000.0
_NEG = -3.0e30


def _body(hf_ref, out_ref):
    hf = hf_ref[...]
    b = hf.shape[0]
    hfp = jnp.concatenate(
        [hf, jnp.full((b, RADIUS), _PAD, jnp.float32)], axis=1
    )
    m = jnp.concatenate(
        [hfp[:, 1:], jnp.full((b, 1), _NEG, jnp.float32)], axis=1
    ) - 1.0
    s = 1
    while s < RADIUS:
        shifted = (
            jnp.concatenate(
                [m[:, s:], jnp.full((b, s), _NEG, jnp.float32)], axis=1
            )
            - float(s)
        )
        m = jnp.maximum(m, shifted)
        s *= 2
    out_ref[...] = jnp.maximum(m[:, :IM_SIZE] - hf, 0.0)


def kernel(height_field):
    return pl.pallas_call(
        _body,
        out_shape=jax.ShapeDtypeStruct((BATCH, IM_SIZE), jnp.float32),
        grid=(4,),
        in_specs=[pl.BlockSpec((BATCH // 4, IM_SIZE), lambda i: (i, 0))],
        out_specs=pl.BlockSpec((BATCH // 4, IM_SIZE), lambda i: (i, 0)),
    )(height_field)
